# baseline (speedup 1.0000x reference)
import jax
import jax.numpy as jnp
from jax.experimental import pallas as pl

D = 64
K = 1024
BLK = 512


def _vq_block(x_ref, w_ref, out_ref, loss_ref):
    i = pl.program_id(0)
    xb = x_ref[...]
    w = w_ref[...]
    wsq = jnp.sum(w * w, axis=1)
    wsum = jnp.sum(w, axis=0)
    s2 = jax.lax.dot_general(xb + xb, w, (((1,), (1,)), ((), ())),
                             preferred_element_type=jnp.float32,
                             precision=jax.lax.Precision.HIGHEST)
    v = s2 - wsq[None, :]
    m = jnp.max(v, axis=1, keepdims=True)
    iota = jax.lax.broadcasted_iota(jnp.int32, v.shape, 1)
    idx = jnp.min(jnp.where(v == m, iota, K), axis=1, keepdims=True)
    lse = m + jnp.log(jnp.sum(jnp.exp(v - m), axis=1, keepdims=True))
    comb = v + (iota == idx).astype(jnp.float32)
    ow = jax.lax.dot_general(comb, w, (((1,), (0,)), ((), ())),
                             preferred_element_type=jnp.float32,
                             precision=jax.lax.Precision.HIGHEST)
    out_ref[...] = 0.5 * (ow - lse * wsum[None, :])
    t = jnp.sum(xb * xb, axis=1, keepdims=True) - m
    part = jnp.sum(t, axis=0, keepdims=True)
    acc = jnp.where(i == 0, 0.0, loss_ref[...])
    loss_ref[...] = acc + part


def kernel(x, codebook):
    xf = x.reshape(-1, D)
    n = xf.shape[0]
    out, loss = pl.pallas_call(
        _vq_block,
        grid=(n // BLK,),
        in_specs=[pl.BlockSpec((BLK, D), lambda i: (i, 0)),
                  pl.BlockSpec((K, D), lambda i: (0, 0))],
        out_specs=[pl.BlockSpec((BLK, D), lambda i: (i, 0)),
                   pl.BlockSpec((1, 1), lambda i: (0, 0))],
        out_shape=[jax.ShapeDtypeStruct((n, D), jnp.float32),
                   jax.ShapeDtypeStruct((1, 1), jnp.float32)],
    )(xf, codebook)
    vq_loss = 1.25 * loss[0, 0] / (n * D)
    return out.reshape(x.shape), vq_loss


# speedup vs baseline: 3.1314x; 3.1314x over previous
import jax
import jax.numpy as jnp
from jax.experimental import pallas as pl

D = 64
K = 1024
BLK = 512


def _vq_block(x_ref, w_ref, out_ref, loss_ref):
    i = pl.program_id(0)
    xb = x_ref[...]
    w = w_ref[...]
    wsq = jnp.sum(w * w, axis=1)
    wsum = jnp.sum(w, axis=0)
    s2 = jax.lax.dot_general(xb + xb, w, (((1,), (1,)), ((), ())),
                             preferred_element_type=jnp.float32,
                             precision=jax.lax.Precision.DEFAULT)
    v = s2 - wsq[None, :]
    m = jnp.max(v, axis=1, keepdims=True)
    iota = jax.lax.broadcasted_iota(jnp.int32, v.shape, 1)
    idx = jnp.min(jnp.where(v == m, iota, K), axis=1, keepdims=True)
    lse = m + jnp.log(jnp.sum(jnp.exp(v - m), axis=1, keepdims=True))
    comb = v + (iota == idx).astype(jnp.float32)
    ow = jax.lax.dot_general(comb, w, (((1,), (0,)), ((), ())),
                             preferred_element_type=jnp.float32,
                             precision=jax.lax.Precision.DEFAULT)
    out_ref[...] = 0.5 * (ow - lse * wsum[None, :])
    t = jnp.sum(xb * xb, axis=1, keepdims=True) - m
    part = jnp.sum(t, axis=0, keepdims=True)
    acc = jnp.where(i == 0, 0.0, loss_ref[...])
    loss_ref[...] = acc + part


def kernel(x, codebook):
    xf = x.reshape(-1, D)
    n = xf.shape[0]
    out, loss = pl.pallas_call(
        _vq_block,
        grid=(n // BLK,),
        in_specs=[pl.BlockSpec((BLK, D), lambda i: (i, 0)),
                  pl.BlockSpec((K, D), lambda i: (0, 0))],
        out_specs=[pl.BlockSpec((BLK, D), lambda i: (i, 0)),
                   pl.BlockSpec((1, 1), lambda i: (0, 0))],
        out_shape=[jax.ShapeDtypeStruct((n, D), jnp.float32),
                   jax.ShapeDtypeStruct((1, 1), jnp.float32)],
    )(xf, codebook)
    vq_loss = 1.25 * loss[0, 0] / (n * D)
    return out.reshape(x.shape), vq_loss


# speedup vs baseline: 3.9660x; 1.2665x over previous
import jax
import jax.numpy as jnp
from jax.experimental import pallas as pl
from jax.experimental.pallas import tpu as pltpu

D = 64
K = 1024
BLK = 512


def _vq_block(x_ref, w_ref, out_ref, loss_ref, w2_ref, wsq_ref, wsh_ref):
    i = pl.program_id(0)

    @pl.when(i == 0)
    def _prep():
        w = w_ref[...]
        w2_ref[...] = w + w
        wsq_ref[...] = jnp.sum(w * w, axis=1)[None, :]
        wsh_ref[...] = 0.5 * jnp.sum(w, axis=0)[None, :]

    xb = x_ref[...]
    w2 = w2_ref[...]
    v = jax.lax.dot_general(xb, w2, (((1,), (1,)), ((), ())),
                            preferred_element_type=jnp.float32) - wsq_ref[...]
    m = jnp.max(v, axis=1, keepdims=True)
    lse = jnp.log(jnp.sum(jnp.exp(v), axis=1, keepdims=True))
    comb = v + (v == m).astype(jnp.float32)
    ow2 = jax.lax.dot_general(comb, w2, (((1,), (0,)), ((), ())),
                              preferred_element_type=jnp.float32)
    out_ref[...] = 0.25 * ow2 - lse * wsh_ref[...]
    t = jnp.sum(xb * xb, axis=1, keepdims=True) - m
    part = jnp.sum(t, axis=0, keepdims=True)
    loss_ref[...] = jnp.where(i == 0, 0.0, loss_ref[...]) + part


def kernel(x, codebook):
    xf = x.reshape(-1, D)
    n = xf.shape[0]
    out, loss = pl.pallas_call(
        _vq_block,
        grid=(n // BLK,),
        in_specs=[pl.BlockSpec((BLK, D), lambda i: (i, 0)),
                  pl.BlockSpec((K, D), lambda i: (0, 0))],
        out_specs=[pl.BlockSpec((BLK, D), lambda i: (i, 0)),
                   pl.BlockSpec((1, 1), lambda i: (0, 0))],
        out_shape=[jax.ShapeDtypeStruct((n, D), jnp.float32),
                   jax.ShapeDtypeStruct((1, 1), jnp.float32)],
        scratch_shapes=[pltpu.VMEM((K, D), jnp.float32),
                        pltpu.VMEM((1, K), jnp.float32),
                        pltpu.VMEM((1, D), jnp.float32)],
    )(xf, codebook)
    vq_loss = 1.25 * loss[0, 0] / (n * D)
    return out.reshape(x.shape), vq_loss


# speedup vs baseline: 4.2213x; 1.0644x over previous
import jax
import jax.numpy as jnp
from jax.experimental import pallas as pl
from jax.experimental.pallas import tpu as pltpu

D = 64
K = 1024
BLK = 1024


def _vq_block(x_ref, w_ref, out_ref, loss_ref, w2_ref, wsq_ref, wsh_ref):
    i = pl.program_id(0)

    @pl.when(i == 0)
    def _prep():
        w = w_ref[...]
        w2_ref[...] = w + w
        wsq_ref[...] = jnp.sum(w * w, axis=1)[None, :]
        wsh_ref[...] = 0.5 * jnp.sum(w, axis=0)[None, :]

    xb = x_ref[...]
    w2 = w2_ref[...]
    v = jax.lax.dot_general(xb, w2, (((1,), (1,)), ((), ())),
                            preferred_element_type=jnp.float32) - wsq_ref[...]
    m = jnp.max(v, axis=1, keepdims=True)
    lse = jnp.log(jnp.sum(jnp.exp(v), axis=1, keepdims=True))
    comb = v + (v == m).astype(jnp.float32)
    ow2 = jax.lax.dot_general(comb, w2, (((1,), (0,)), ((), ())),
                              preferred_element_type=jnp.float32)
    out_ref[...] = 0.25 * ow2 - lse * wsh_ref[...]
    t = jnp.sum(xb * xb, axis=1, keepdims=True) - m
    part = jnp.sum(t, axis=0, keepdims=True)
    loss_ref[...] = jnp.where(i == 0, 0.0, loss_ref[...]) + part


def kernel(x, codebook):
    xf = x.reshape(-1, D)
    n = xf.shape[0]
    out, loss = pl.pallas_call(
        _vq_block,
        grid=(n // BLK,),
        in_specs=[pl.BlockSpec((BLK, D), lambda i: (i, 0)),
                  pl.BlockSpec((K, D), lambda i: (0, 0))],
        out_specs=[pl.BlockSpec((BLK, D), lambda i: (i, 0)),
                   pl.BlockSpec((1, 1), lambda i: (0, 0))],
        out_shape=[jax.ShapeDtypeStruct((n, D), jnp.float32),
                   jax.ShapeDtypeStruct((1, 1), jnp.float32)],
        scratch_shapes=[pltpu.VMEM((K, D), jnp.float32),
                        pltpu.VMEM((1, K), jnp.float32),
                        pltpu.VMEM((1, D), jnp.float32)],
    )(xf, codebook)
    vq_loss = 1.25 * loss[0, 0] / (n * D)
    return out.reshape(x.shape), vq_loss


# speedup vs baseline: 4.3981x; 1.0419x over previous
import jax
import jax.numpy as jnp
from jax.experimental import pallas as pl
from jax.experimental.pallas import tpu as pltpu

D = 64
K = 1024
BLK = 2048


def _vq_block(x_ref, w_ref, out_ref, loss_ref, w2_ref, wsq_ref, wsh_ref):
    i = pl.program_id(0)

    @pl.when(i == 0)
    def _prep():
        w = w_ref[...]
        w2_ref[...] = w + w
        wsq_ref[...] = jnp.sum(w * w, axis=1)[None, :]
        wsh_ref[...] = 0.5 * jnp.sum(w, axis=0)[None, :]

    xb = x_ref[...]
    w2 = w2_ref[...]
    v = jax.lax.dot_general(xb, w2, (((1,), (1,)), ((), ())),
                            preferred_element_type=jnp.float32) - wsq_ref[...]
    m = jnp.max(v, axis=1, keepdims=True)
    lse = jnp.log(jnp.sum(jnp.exp(v), axis=1, keepdims=True))
    comb = v + (v == m).astype(jnp.float32)
    ow2 = jax.lax.dot_general(comb, w2, (((1,), (0,)), ((), ())),
                              preferred_element_type=jnp.float32)
    out_ref[...] = 0.25 * ow2 - lse * wsh_ref[...]
    t = jnp.sum(xb * xb, axis=1, keepdims=True) - m
    part = jnp.sum(t, axis=0, keepdims=True)
    loss_ref[...] = jnp.where(i == 0, 0.0, loss_ref[...]) + part


def kernel(x, codebook):
    xf = x.reshape(-1, D)
    n = xf.shape[0]
    out, loss = pl.pallas_call(
        _vq_block,
        grid=(n // BLK,),
        in_specs=[pl.BlockSpec((BLK, D), lambda i: (i, 0)),
                  pl.BlockSpec((K, D), lambda i: (0, 0))],
        out_specs=[pl.BlockSpec((BLK, D), lambda i: (i, 0)),
                   pl.BlockSpec((1, 1), lambda i: (0, 0))],
        out_shape=[jax.ShapeDtypeStruct((n, D), jnp.float32),
                   jax.ShapeDtypeStruct((1, 1), jnp.float32)],
        scratch_shapes=[pltpu.VMEM((K, D), jnp.float32),
                        pltpu.VMEM((1, K), jnp.float32),
                        pltpu.VMEM((1, D), jnp.float32)],
    )(xf, codebook)
    vq_loss = 1.25 * loss[0, 0] / (n * D)
    return out.reshape(x.shape), vq_loss
